# Initial kernel scaffold; baseline (speedup 1.0000x reference)
import jax
import jax.numpy as jnp
from jax.experimental import pallas as pl


def kernel(x, edge_index, batch, W_vertex, b_vertex, W_edge1, b_edge1, W_x, b_x, W_edge2, b_edge2, W_out, b_out):
    raise NotImplementedError("write your pallas kernel here")


# speedup vs baseline: 22.7398x; 22.7398x over previous
import functools

import jax
import jax.numpy as jnp
from jax import lax
from jax.experimental import pallas as pl
from jax.experimental.pallas import tpu as pltpu
from jax.experimental.pallas import tpu_sc as plsc

N = 100000
E = 1600000
BS = 2048
NB = 49
NPAD = NB * BS
NW = 32
MB = 128
K = 8
TPT = 392
EPAD = NW * TPT * MB


def _edge_pass_grid(table, srcb, dstb, feat):
  vec = (NPAD, feat) if feat else (NPAD,)
  val_shape = (MB, feat) if feat else (MB,)
  rows_pt = NPAD // 16
  nchunk = 16 if feat else 8
  zchunk = rows_pt // nchunk

  mesh = plsc.VectorSubcoreMesh(core_axis_name="c", subcore_axis_name="s")

  @functools.partial(
      pl.kernel,
      out_type=jax.ShapeDtypeStruct((2,) + vec, jnp.float32),
      mesh=mesh,
      compiler_params=pltpu.CompilerParams(use_tc_tiling_on_sc=False),
      scratch_types=(
          [pltpu.VMEM_SHARED(vec, jnp.float32)]
          + [pltpu.VMEM((MB,), jnp.int32) for _ in range(2 * K)]
          + [pltpu.VMEM(val_shape, jnp.float32) for _ in range(K)]
          + [pltpu.VMEM((zchunk, feat) if feat else (zchunk,), jnp.float32)]
          + [pltpu.SemaphoreType.DMA]
      ),
  )
  def kfn(table_ref, src_ref, dst_ref, out_ref, acc, *rest):
    sidx = rest[:K]
    didx = rest[K:2 * K]
    vals = rest[2 * K:3 * K]
    bounce = rest[3 * K]
    gsem = rest[3 * K + 1]
    cid = lax.axis_index("c")
    sid = lax.axis_index("s")
    wid = cid * 16 + sid

    def _zero(i, _):
      if feat:
        bounce[i, :] = jnp.zeros((16,), jnp.float32)
      else:
        bounce[pl.ds(i * 16, 16)] = jnp.zeros((16,), jnp.float32)
      return _
    lax.fori_loop(0, zchunk if feat else zchunk // 16, _zero, None)
    row0 = sid * rows_pt
    for j in range(nchunk):
      pltpu.sync_copy(bounce, acc.at[pl.ds(row0 + j * zchunk, zchunk)])
    plsc.subcore_barrier()

    ebase = wid * (TPT * MB)

    def _blk(b, _):
      off0 = ebase + b * (K * MB)
      descs = []
      for k in range(K):
        off = off0 + k * MB
        pltpu.sync_copy(src_ref.at[pl.ds(off, MB)], sidx[k])
        pltpu.sync_copy(dst_ref.at[pl.ds(off, MB)], didx[k])
        descs.append(pltpu.async_copy(table_ref.at[sidx[k]], vals[k], gsem))
      for k in range(K):
        descs[k].wait()
        pltpu.sync_copy(
            vals[k],
            acc.at[plsc.Indices(didx[k], ignored_value=-1)],
            add=True,
        )
      return _
    lax.fori_loop(0, TPT // K, _blk, None)
    plsc.subcore_barrier()

    for j in range(nchunk):
      r = row0 + j * zchunk
      pltpu.sync_copy(acc.at[pl.ds(r, zchunk)], bounce)
      pltpu.sync_copy(bounce, out_ref.at[cid, pl.ds(r, zchunk)])

  return kfn(table, srcb, dstb)


def _node_stage1(xpad, pa, pb, wvecs, consts):
  def body(x_ref, pa_ref, pb_ref, w_ref, c_ref, s_ref, a0_ref, inv_ref):
    xb = x_ref[...]
    p = pa_ref[...] + pb_ref[...]
    deg = p[:, 11]
    mask = deg > 0.5
    inv = jnp.where(mask, 1.0 / jnp.where(mask, deg, 1.0), 0.0)
    w = w_ref[...]
    c = c_ref[...]
    hu1 = jnp.sum(xb * w[0:1, :], axis=1) + c[0, 0] \
        + jnp.sum(p * w[1:2, :], axis=1) * inv
    hu2 = jnp.sum(xb * w[2:3, :], axis=1) + c[0, 1] \
        + jnp.sum(p * w[3:4, :], axis=1) * inv
    hu1 = jnp.where(mask, hu1, 0.0)
    hu2 = jnp.where(mask, hu2, 0.0)
    s_ref[...] = (hu2 + c[0, 3]).reshape(1, 1, BS)
    a0_ref[...] = (hu1 + c[0, 2]).reshape(1, 1, BS)
    inv_ref[...] = inv.reshape(1, 1, BS)

  out = jax.ShapeDtypeStruct((NB, 1, BS), jnp.float32)
  return pl.pallas_call(
      body,
      grid=(NB,),
      in_specs=[
          pl.BlockSpec((BS, 16), lambda i: (i, 0)),
          pl.BlockSpec((BS, 16), lambda i: (i, 0)),
          pl.BlockSpec((BS, 16), lambda i: (i, 0)),
          pl.BlockSpec((4, 16), lambda i: (0, 0)),
          pl.BlockSpec((1, 16), lambda i: (0, 0)),
      ],
      out_specs=[
          pl.BlockSpec((1, 1, BS), lambda i: (i, 0, 0)),
          pl.BlockSpec((1, 1, BS), lambda i: (i, 0, 0)),
          pl.BlockSpec((1, 1, BS), lambda i: (i, 0, 0)),
      ],
      out_shape=[out, out, out],
  )(xpad.reshape(NPAD, 16), pa, pb, wvecs, consts)


def _node_stage2(a0, inv, qa, qb, batch2, consts):
  def body(a0_ref, inv_ref, qa_ref, qb_ref, b_ref, c_ref, out_ref):
    i = pl.program_id(0)
    inv = inv_ref[...].reshape(1, BS)
    val = jnp.where(
        inv > 0.0,
        a0_ref[...].reshape(1, BS)
        + (qa_ref[...].reshape(1, BS) + qb_ref[...].reshape(1, BS)) * inv,
        0.0,
    ) + c_ref[0, 0]
    bids = b_ref[...].reshape(1, BS)
    seg = lax.broadcasted_iota(jnp.int32, (64, BS), 0)
    masked = jnp.where(bids == seg, val, -jnp.inf)
    part = jnp.max(masked, axis=1, keepdims=True)

    @pl.when(i == 0)
    def _():
      out_ref[...] = jnp.full((64, 1), -jnp.inf, jnp.float32)
    out_ref[...] = jnp.maximum(out_ref[...], part)

  spec = pl.BlockSpec((1, 1, BS), lambda i: (i, 0, 0))
  return pl.pallas_call(
      body,
      grid=(NB,),
      in_specs=[spec, spec, spec, spec, spec,
                pl.BlockSpec((1, 16), lambda i: (0, 0))],
      out_specs=pl.BlockSpec((64, 1), lambda i: (0, 0)),
      out_shape=jax.ShapeDtypeStruct((64, 1), jnp.float32),
  )(a0, inv, qa, qb, batch2, consts)


def kernel(x, edge_index, batch, W_vertex, b_vertex, W_edge1, b_edge1,
           W_x, b_x, W_edge2, b_edge2, W_out, b_out):
  f32 = jnp.float32
  hdot = functools.partial(jnp.dot, precision=jax.lax.Precision.HIGHEST)
  wo = W_out[:, 0]
  w1 = hdot(W_edge2[:64], wo)
  w2 = hdot(W_edge2[64:], wo)
  u1 = hdot(W_x, w1)
  u2 = hdot(W_x, w2)

  def vecs(u):
    a1u = hdot(W_edge1[:64], u)
    b1u = hdot(W_edge1[64:128], u)
    p = hdot(W_vertex, a1u).at[9:11].add(hdot(W_edge1[128:130], u))
    q = hdot(W_vertex, b1u).at[9:11].add(hdot(W_edge1[130:132], u))
    pv = jnp.concatenate([p, jnp.zeros((5,), f32)])
    qv = jnp.concatenate([q, hdot(b_vertex, b1u)[None],
                          jnp.zeros((4,), f32)])
    cu = hdot(b_vertex, a1u) + hdot(b_edge1, u)
    return pv, qv, cu

  pv1, qv1, c1 = vecs(u1)
  pv2, qv2, c2 = vecs(u2)
  wvecs = jnp.stack([pv1, qv1, pv2, qv2])
  e1g = hdot(b_x, w1) + hdot(b_edge2, wo)
  e2 = hdot(b_x, w2)
  consts1 = jnp.zeros((1, 16), f32).at[0, :4].set(
      jnp.stack([c1, c2, e1g, e2]))
  consts2 = jnp.zeros((1, 16), f32).at[0, 0].set(b_out[0])

  xpad = jnp.concatenate(
      [x, jnp.ones((N, 1), f32), jnp.zeros((N, 4), f32)], axis=1)
  xpad = jnp.concatenate([xpad, jnp.zeros((NPAD - N, 16), f32)])
  src = jnp.concatenate(
      [edge_index[0], jnp.zeros((EPAD - E,), jnp.int32)])
  dst = jnp.concatenate(
      [edge_index[1], jnp.full((EPAD - E,), -1, jnp.int32)])
  batch2 = jnp.concatenate(
      [batch, jnp.full((NPAD - N,), 64, jnp.int32)]).reshape(NB, 1, BS)

  part1 = _edge_pass_grid(xpad, src, dst, 16)
  s2, a0, inv = _node_stage1(xpad, part1[0], part1[1], wvecs, consts1)
  part2 = _edge_pass_grid(s2.reshape(NPAD), src, dst, 0)
  return _node_stage2(a0, inv, part2[0].reshape(NB, 1, BS),
                      part2[1].reshape(NB, 1, BS), batch2, consts2)


# speedup vs baseline: 34.2458x; 1.5060x over previous
import functools

import jax
import jax.numpy as jnp
from jax import lax
from jax.experimental import pallas as pl
from jax.experimental.pallas import tpu as pltpu
from jax.experimental.pallas import tpu_sc as plsc

N = 100000
E = 1600000
BS = 2048
NB = 49
NPAD = NB * BS
NW = 32
MB = 128
K = 8
TPT = 392
EPAD = NW * TPT * MB


def _edge_pass_grid(table, srcb, dstb, feat):
  vec = (NPAD, feat) if feat else (NPAD,)
  val_shape = (MB, feat) if feat else (MB,)
  rows_pb = K
  rows_pt = NPAD // 16
  nchunk = 16 if feat else 8
  zchunk = rows_pt // nchunk

  mesh = plsc.VectorSubcoreMesh(core_axis_name="c", subcore_axis_name="s")

  @functools.partial(
      pl.kernel,
      out_type=jax.ShapeDtypeStruct((2,) + vec, jnp.float32),
      mesh=mesh,
      compiler_params=pltpu.CompilerParams(use_tc_tiling_on_sc=False),
      scratch_types=(
          [pltpu.VMEM_SHARED(vec, jnp.float32)]
          + [pltpu.VMEM((K, MB), jnp.int32) for _ in range(2)]
          + [pltpu.VMEM(val_shape, jnp.float32) for _ in range(K)]
          + [pltpu.VMEM((zchunk, feat) if feat else (zchunk,), jnp.float32)]
          + [pltpu.SemaphoreType.DMA, pltpu.SemaphoreType.DMA]
      ),
  )
  def kfn(table_ref, src_ref, dst_ref, out_ref, acc, *rest):
    sidx, didx = rest[0], rest[1]
    vals = rest[2:2 + K]
    bounce = rest[2 + K]
    gsem = rest[3 + K]
    ssem = rest[4 + K]
    cid = lax.axis_index("c")
    sid = lax.axis_index("s")
    wid = cid * 16 + sid

    def _zero(i, _):
      if feat:
        bounce[i, :] = jnp.zeros((16,), jnp.float32)
      else:
        bounce[pl.ds(i * 16, 16)] = jnp.zeros((16,), jnp.float32)
      return _
    lax.fori_loop(0, zchunk if feat else zchunk // 16, _zero, None)
    row0 = sid * rows_pt
    for j in range(nchunk):
      pltpu.sync_copy(bounce, acc.at[pl.ds(row0 + j * zchunk, zchunk)])
    plsc.subcore_barrier()

    rbase = wid * TPT

    def _blk(b, _):
      r0 = rbase + b * rows_pb
      pltpu.sync_copy(src_ref.at[pl.ds(r0, rows_pb)], sidx)
      pltpu.sync_copy(dst_ref.at[pl.ds(r0, rows_pb)], didx)
      descs = []
      for k in range(K):
        descs.append(
            pltpu.async_copy(table_ref.at[sidx.at[k]], vals[k], gsem))
      sdescs = []
      for k in range(K):
        descs[k].wait()
        sdescs.append(pltpu.async_copy(
            vals[k],
            acc.at[plsc.Indices(didx.at[k], ignored_value=-1)],
            ssem,
            add=True,
        ))
      for k in range(K):
        sdescs[k].wait()
      return _
    lax.fori_loop(0, TPT // K, _blk, None)
    plsc.subcore_barrier()

    for j in range(nchunk):
      r = row0 + j * zchunk
      pltpu.sync_copy(acc.at[pl.ds(r, zchunk)], bounce)
      pltpu.sync_copy(bounce, out_ref.at[cid, pl.ds(r, zchunk)])

  return kfn(table, srcb, dstb)


def _node_stage1(xpad, pa, pb, wvecs, consts):
  def body(x_ref, pa_ref, pb_ref, w_ref, c_ref, s_ref, a0_ref, inv_ref):
    xb = x_ref[...]
    p = pa_ref[...] + pb_ref[...]
    deg = p[:, 11]
    mask = deg > 0.5
    inv = jnp.where(mask, 1.0 / jnp.where(mask, deg, 1.0), 0.0)
    w = w_ref[...]
    c = c_ref[...]
    hu1 = jnp.sum(xb * w[0:1, :], axis=1) + c[0, 0] \
        + jnp.sum(p * w[1:2, :], axis=1) * inv
    hu2 = jnp.sum(xb * w[2:3, :], axis=1) + c[0, 1] \
        + jnp.sum(p * w[3:4, :], axis=1) * inv
    hu1 = jnp.where(mask, hu1, 0.0)
    hu2 = jnp.where(mask, hu2, 0.0)
    s_ref[...] = (hu2 + c[0, 3]).reshape(1, 1, BS)
    a0_ref[...] = (hu1 + c[0, 2]).reshape(1, 1, BS)
    inv_ref[...] = inv.reshape(1, 1, BS)

  out = jax.ShapeDtypeStruct((NB, 1, BS), jnp.float32)
  return pl.pallas_call(
      body,
      grid=(NB,),
      in_specs=[
          pl.BlockSpec((BS, 16), lambda i: (i, 0)),
          pl.BlockSpec((BS, 16), lambda i: (i, 0)),
          pl.BlockSpec((BS, 16), lambda i: (i, 0)),
          pl.BlockSpec((4, 16), lambda i: (0, 0)),
          pl.BlockSpec((1, 16), lambda i: (0, 0)),
      ],
      out_specs=[
          pl.BlockSpec((1, 1, BS), lambda i: (i, 0, 0)),
          pl.BlockSpec((1, 1, BS), lambda i: (i, 0, 0)),
          pl.BlockSpec((1, 1, BS), lambda i: (i, 0, 0)),
      ],
      out_shape=[out, out, out],
  )(xpad.reshape(NPAD, 16), pa, pb, wvecs, consts)


def _node_stage2(a0, inv, qa, qb, batch2, consts):
  def body(a0_ref, inv_ref, qa_ref, qb_ref, b_ref, c_ref, out_ref):
    i = pl.program_id(0)
    inv = inv_ref[...].reshape(1, BS)
    val = jnp.where(
        inv > 0.0,
        a0_ref[...].reshape(1, BS)
        + (qa_ref[...].reshape(1, BS) + qb_ref[...].reshape(1, BS)) * inv,
        0.0,
    ) + c_ref[0, 0]
    bids = b_ref[...].reshape(1, BS)
    seg = lax.broadcasted_iota(jnp.int32, (64, BS), 0)
    masked = jnp.where(bids == seg, val, -jnp.inf)
    part = jnp.max(masked, axis=1, keepdims=True)

    @pl.when(i == 0)
    def _():
      out_ref[...] = jnp.full((64, 1), -jnp.inf, jnp.float32)
    out_ref[...] = jnp.maximum(out_ref[...], part)

  spec = pl.BlockSpec((1, 1, BS), lambda i: (i, 0, 0))
  return pl.pallas_call(
      body,
      grid=(NB,),
      in_specs=[spec, spec, spec, spec, spec,
                pl.BlockSpec((1, 16), lambda i: (0, 0))],
      out_specs=pl.BlockSpec((64, 1), lambda i: (0, 0)),
      out_shape=jax.ShapeDtypeStruct((64, 1), jnp.float32),
  )(a0, inv, qa, qb, batch2, consts)


def kernel(x, edge_index, batch, W_vertex, b_vertex, W_edge1, b_edge1,
           W_x, b_x, W_edge2, b_edge2, W_out, b_out):
  f32 = jnp.float32
  hdot = functools.partial(jnp.dot, precision=jax.lax.Precision.HIGHEST)
  wo = W_out[:, 0]
  w1 = hdot(W_edge2[:64], wo)
  w2 = hdot(W_edge2[64:], wo)
  u1 = hdot(W_x, w1)
  u2 = hdot(W_x, w2)

  def vecs(u):
    a1u = hdot(W_edge1[:64], u)
    b1u = hdot(W_edge1[64:128], u)
    p = hdot(W_vertex, a1u).at[9:11].add(hdot(W_edge1[128:130], u))
    q = hdot(W_vertex, b1u).at[9:11].add(hdot(W_edge1[130:132], u))
    pv = jnp.concatenate([p, jnp.zeros((5,), f32)])
    qv = jnp.concatenate([q, hdot(b_vertex, b1u)[None],
                          jnp.zeros((4,), f32)])
    cu = hdot(b_vertex, a1u) + hdot(b_edge1, u)
    return pv, qv, cu

  pv1, qv1, c1 = vecs(u1)
  pv2, qv2, c2 = vecs(u2)
  wvecs = jnp.stack([pv1, qv1, pv2, qv2])
  e1g = hdot(b_x, w1) + hdot(b_edge2, wo)
  e2 = hdot(b_x, w2)
  consts1 = jnp.zeros((1, 16), f32).at[0, :4].set(
      jnp.stack([c1, c2, e1g, e2]))
  consts2 = jnp.zeros((1, 16), f32).at[0, 0].set(b_out[0])

  xpad = jnp.concatenate(
      [x, jnp.ones((N, 1), f32), jnp.zeros((N, 4), f32)], axis=1)
  xpad = jnp.concatenate([xpad, jnp.zeros((NPAD - N, 16), f32)])
  src = jnp.concatenate(
      [edge_index[0], jnp.zeros((EPAD - E,), jnp.int32)]).reshape(-1, MB)
  dst = jnp.concatenate(
      [edge_index[1], jnp.full((EPAD - E,), -1, jnp.int32)]).reshape(-1, MB)
  batch2 = jnp.concatenate(
      [batch, jnp.full((NPAD - N,), 64, jnp.int32)]).reshape(NB, 1, BS)

  part1 = _edge_pass_grid(xpad, src, dst, 16)
  s2, a0, inv = _node_stage1(xpad, part1[0], part1[1], wvecs, consts1)
  part2 = _edge_pass_grid(s2.reshape(NPAD), src, dst, 0)
  return _node_stage2(a0, inv, part2[0].reshape(NB, 1, BS),
                      part2[1].reshape(NB, 1, BS), batch2, consts2)


# speedup vs baseline: 38.2193x; 1.1160x over previous
import functools

import jax
import jax.numpy as jnp
from jax import lax
from jax.experimental import pallas as pl
from jax.experimental.pallas import tpu as pltpu
from jax.experimental.pallas import tpu_sc as plsc

N = 100000
E = 1600000
BS = 2048
NB = 49
NPAD = NB * BS
NW = 32
MB = 128
K = 8
TPT = 392
EPAD = NW * TPT * MB


def _edge_pass_grid(table, srcb, dstb, feat):
  vec = (NPAD, feat) if feat else (NPAD,)
  val_shape = (MB, feat) if feat else (MB,)
  rows_pb = K
  rows_pt = NPAD // 16
  nchunk = 16 if feat else 8
  zchunk = rows_pt // nchunk

  mesh = plsc.VectorSubcoreMesh(core_axis_name="c", subcore_axis_name="s")

  @functools.partial(
      pl.kernel,
      out_type=jax.ShapeDtypeStruct((2,) + vec, jnp.float32),
      mesh=mesh,
      compiler_params=pltpu.CompilerParams(use_tc_tiling_on_sc=False),
      scratch_types=(
          [pltpu.VMEM_SHARED(vec, jnp.float32)]
          + [pltpu.VMEM((2, K, MB), jnp.int32) for _ in range(2)]
          + [pltpu.VMEM(val_shape, jnp.float32) for _ in range(K)]
          + [pltpu.VMEM((zchunk, feat) if feat else (zchunk,), jnp.float32)]
          + [pltpu.SemaphoreType.DMA] * 3
      ),
  )
  def kfn(table_ref, src_ref, dst_ref, out_ref, acc, *rest):
    sidx, didx = rest[0], rest[1]
    vals = rest[2:2 + K]
    bounce = rest[2 + K]
    isem = rest[3 + K]
    gsem = rest[4 + K]
    ssem = rest[5 + K]
    cid = lax.axis_index("c")
    sid = lax.axis_index("s")
    wid = cid * 16 + sid

    def _zero(i, _):
      if feat:
        bounce[i, :] = jnp.zeros((16,), jnp.float32)
      else:
        bounce[pl.ds(i * 16, 16)] = jnp.zeros((16,), jnp.float32)
      return _
    lax.fori_loop(0, zchunk if feat else zchunk // 16, _zero, None)
    row0 = sid * rows_pt
    for j in range(nchunk):
      pltpu.sync_copy(bounce, acc.at[pl.ds(row0 + j * zchunk, zchunk)])
    plsc.subcore_barrier()

    rbase = wid * TPT
    nblk = TPT // K
    pltpu.async_copy(src_ref.at[pl.ds(rbase, rows_pb)], sidx.at[0], isem)
    pltpu.async_copy(dst_ref.at[pl.ds(rbase, rows_pb)], didx.at[0], isem)

    def _blk(b, _):
      p = lax.rem(b, 2)
      r0 = rbase + b * rows_pb
      pltpu.make_async_copy(
          src_ref.at[pl.ds(r0, rows_pb)], sidx.at[p], isem).wait()
      pltpu.make_async_copy(
          dst_ref.at[pl.ds(r0, rows_pb)], didx.at[p], isem).wait()

      @pl.when(b < nblk - 1)
      def _():
        r1 = r0 + rows_pb
        pltpu.async_copy(src_ref.at[pl.ds(r1, rows_pb)], sidx.at[1 - p], isem)
        pltpu.async_copy(dst_ref.at[pl.ds(r1, rows_pb)], didx.at[1 - p], isem)

      descs = []
      for k in range(K):
        descs.append(
            pltpu.async_copy(table_ref.at[sidx.at[p, k]], vals[k], gsem))
      sdescs = []
      for k in range(K):
        descs[k].wait()
        sdescs.append(pltpu.async_copy(
            vals[k],
            acc.at[plsc.Indices(didx.at[p, k], ignored_value=-1)],
            ssem,
            add=True,
        ))
      for k in range(K):
        sdescs[k].wait()
      return _
    lax.fori_loop(0, nblk, _blk, None)
    plsc.subcore_barrier()

    for j in range(nchunk):
      r = row0 + j * zchunk
      pltpu.sync_copy(acc.at[pl.ds(r, zchunk)], bounce)
      pltpu.sync_copy(bounce, out_ref.at[cid, pl.ds(r, zchunk)])

  return kfn(table, srcb, dstb)


def _node_stage1(xpad, pa, pb, wvecs, consts):
  def body(x_ref, pa_ref, pb_ref, w_ref, c_ref, s_ref, a0_ref, inv_ref):
    xb = x_ref[...]
    p = pa_ref[...] + pb_ref[...]
    deg = p[:, 11]
    mask = deg > 0.5
    inv = jnp.where(mask, 1.0 / jnp.where(mask, deg, 1.0), 0.0)
    w = w_ref[...]
    c = c_ref[...]
    hu1 = jnp.sum(xb * w[0:1, :], axis=1) + c[0, 0] \
        + jnp.sum(p * w[1:2, :], axis=1) * inv
    hu2 = jnp.sum(xb * w[2:3, :], axis=1) + c[0, 1] \
        + jnp.sum(p * w[3:4, :], axis=1) * inv
    hu1 = jnp.where(mask, hu1, 0.0)
    hu2 = jnp.where(mask, hu2, 0.0)
    s_ref[...] = (hu2 + c[0, 3]).reshape(1, 1, BS)
    a0_ref[...] = (hu1 + c[0, 2]).reshape(1, 1, BS)
    inv_ref[...] = inv.reshape(1, 1, BS)

  out = jax.ShapeDtypeStruct((NB, 1, BS), jnp.float32)
  return pl.pallas_call(
      body,
      grid=(NB,),
      in_specs=[
          pl.BlockSpec((BS, 16), lambda i: (i, 0)),
          pl.BlockSpec((BS, 16), lambda i: (i, 0)),
          pl.BlockSpec((BS, 16), lambda i: (i, 0)),
          pl.BlockSpec((4, 16), lambda i: (0, 0)),
          pl.BlockSpec((1, 16), lambda i: (0, 0)),
      ],
      out_specs=[
          pl.BlockSpec((1, 1, BS), lambda i: (i, 0, 0)),
          pl.BlockSpec((1, 1, BS), lambda i: (i, 0, 0)),
          pl.BlockSpec((1, 1, BS), lambda i: (i, 0, 0)),
      ],
      out_shape=[out, out, out],
  )(xpad.reshape(NPAD, 16), pa, pb, wvecs, consts)


def _node_stage2(a0, inv, qa, qb, batch2, consts):
  def body(a0_ref, inv_ref, qa_ref, qb_ref, b_ref, c_ref, out_ref):
    i = pl.program_id(0)
    inv = inv_ref[...].reshape(1, BS)
    val = jnp.where(
        inv > 0.0,
        a0_ref[...].reshape(1, BS)
        + (qa_ref[...].reshape(1, BS) + qb_ref[...].reshape(1, BS)) * inv,
        0.0,
    ) + c_ref[0, 0]
    bids = b_ref[...].reshape(1, BS)
    seg = lax.broadcasted_iota(jnp.int32, (64, BS), 0)
    masked = jnp.where(bids == seg, val, -jnp.inf)
    part = jnp.max(masked, axis=1, keepdims=True)

    @pl.when(i == 0)
    def _():
      out_ref[...] = jnp.full((64, 1), -jnp.inf, jnp.float32)
    out_ref[...] = jnp.maximum(out_ref[...], part)

  spec = pl.BlockSpec((1, 1, BS), lambda i: (i, 0, 0))
  return pl.pallas_call(
      body,
      grid=(NB,),
      in_specs=[spec, spec, spec, spec, spec,
                pl.BlockSpec((1, 16), lambda i: (0, 0))],
      out_specs=pl.BlockSpec((64, 1), lambda i: (0, 0)),
      out_shape=jax.ShapeDtypeStruct((64, 1), jnp.float32),
  )(a0, inv, qa, qb, batch2, consts)


def kernel(x, edge_index, batch, W_vertex, b_vertex, W_edge1, b_edge1,
           W_x, b_x, W_edge2, b_edge2, W_out, b_out):
  f32 = jnp.float32
  hdot = functools.partial(jnp.dot, precision=jax.lax.Precision.HIGHEST)
  wo = W_out[:, 0]
  w1 = hdot(W_edge2[:64], wo)
  w2 = hdot(W_edge2[64:], wo)
  u1 = hdot(W_x, w1)
  u2 = hdot(W_x, w2)

  def vecs(u):
    a1u = hdot(W_edge1[:64], u)
    b1u = hdot(W_edge1[64:128], u)
    p = hdot(W_vertex, a1u).at[9:11].add(hdot(W_edge1[128:130], u))
    q = hdot(W_vertex, b1u).at[9:11].add(hdot(W_edge1[130:132], u))
    pv = jnp.concatenate([p, jnp.zeros((5,), f32)])
    qv = jnp.concatenate([q, hdot(b_vertex, b1u)[None],
                          jnp.zeros((4,), f32)])
    cu = hdot(b_vertex, a1u) + hdot(b_edge1, u)
    return pv, qv, cu

  pv1, qv1, c1 = vecs(u1)
  pv2, qv2, c2 = vecs(u2)
  wvecs = jnp.stack([pv1, qv1, pv2, qv2])
  e1g = hdot(b_x, w1) + hdot(b_edge2, wo)
  e2 = hdot(b_x, w2)
  consts1 = jnp.zeros((1, 16), f32).at[0, :4].set(
      jnp.stack([c1, c2, e1g, e2]))
  consts2 = jnp.zeros((1, 16), f32).at[0, 0].set(b_out[0])

  xpad = jnp.concatenate(
      [x, jnp.ones((N, 1), f32), jnp.zeros((N, 4), f32)], axis=1)
  xpad = jnp.concatenate([xpad, jnp.zeros((NPAD - N, 16), f32)])
  src = jnp.concatenate(
      [edge_index[0], jnp.zeros((EPAD - E,), jnp.int32)]).reshape(-1, MB)
  dst = jnp.concatenate(
      [edge_index[1], jnp.full((EPAD - E,), -1, jnp.int32)]).reshape(-1, MB)
  batch2 = jnp.concatenate(
      [batch, jnp.full((NPAD - N,), 64, jnp.int32)]).reshape(NB, 1, BS)

  part1 = _edge_pass_grid(xpad, src, dst, 16)
  s2, a0, inv = _node_stage1(xpad, part1[0], part1[1], wvecs, consts1)
  part2 = _edge_pass_grid(s2.reshape(NPAD), src, dst, 0)
  return _node_stage2(a0, inv, part2[0].reshape(NB, 1, BS),
                      part2[1].reshape(NB, 1, BS), batch2, consts2)
